# Initial kernel scaffold; baseline (speedup 1.0000x reference)
import jax
import jax.numpy as jnp
from jax.experimental import pallas as pl


def kernel(x, conv_w, conv_b):
    raise NotImplementedError("write your pallas kernel here")


# speedup vs baseline: 18.9516x; 18.9516x over previous
import functools

import jax
import jax.numpy as jnp
from jax import lax
from jax.experimental import pallas as pl
from jax.experimental.pallas import tpu as pltpu

EPS = 1e-05
B, C, H, W = 2, 96, 384, 384
DH, DW = 192, 192
NPIX = H * W
NDST = DH * DW
HB = 16
PB = HB * W
NHB = H // HB
CK = 52


def _tc1_body(x_ref, w_ref, b_ref, off_ref, dest_ref, lin_ref, xw0_ref, xw1_ref):
    b = pl.program_id(0)
    h = pl.program_id(1)
    x2 = x_ref[0]
    w = w_ref[...]
    oa = lax.dot_general(w, x2, (((1,), (0,)), ((), ())),
                         preferred_element_type=jnp.float32)
    oa = oa + b_ref[...]
    pix = h * PB + lax.broadcasted_iota(jnp.int32, (1, PB), 1)
    rows = (pix // W).astype(jnp.float32)
    cols = (pix % W).astype(jnp.float32)
    gy = rows / jnp.float32(H)
    gx = cols / jnp.float32(W)
    dy = jnp.floor(jnp.clip(gy + oa[0:1], 0.0, 1.0 - EPS) * DH)
    dx = jnp.floor(jnp.clip(gx + oa[1:2], 0.0, 1.0 - EPS) * DW)
    lin = dy * DW + dx
    off_ref[0] = oa[0:2]
    chan = lax.broadcasted_iota(jnp.int32, (C, 1), 0).astype(jnp.float32) * jnp.float32(NDST)
    dest_ref[0] = lin + chan + b.astype(jnp.float32) * jnp.float32(C * NDST)
    lin_ref[0] = lin.astype(jnp.int32)
    att = jnp.exp(oa[2:3])
    xw_full = jnp.concatenate(
        [x2 * att, att, jnp.zeros((2 * CK - C - 1, PB), jnp.float32)], axis=0)
    xwt = xw_full.T
    xw0_ref[0] = xwt[:, :CK]
    xw1_ref[0] = xwt[:, CK:]


@jax.jit
def _tc1(x, conv_w, conv_b):
    return pl.pallas_call(
        _tc1_body,
        grid=(B, NHB),
        in_specs=[
            pl.BlockSpec((1, C, PB), lambda b, h: (b, 0, h)),
            pl.BlockSpec((3, C), lambda b, h: (0, 0)),
            pl.BlockSpec((3, 1), lambda b, h: (0, 0)),
        ],
        out_specs=[
            pl.BlockSpec((1, 2, PB), lambda b, h: (b, 0, h)),
            pl.BlockSpec((1, C, PB), lambda b, h: (b, 0, h)),
            pl.BlockSpec((1, 1, PB), lambda b, h: (b * NHB + h, 0, 0)),
            pl.BlockSpec((1, PB, CK), lambda b, h: (b, h, 0)),
            pl.BlockSpec((1, PB, CK), lambda b, h: (b, h, 0)),
        ],
        out_shape=[
            jax.ShapeDtypeStruct((B, 2, NPIX), jnp.float32),
            jax.ShapeDtypeStruct((B, C, NPIX), jnp.float32),
            jax.ShapeDtypeStruct((B * NHB, 1, PB), jnp.int32),
            jax.ShapeDtypeStruct((B, NPIX, CK), jnp.float32),
            jax.ShapeDtypeStruct((B, NPIX, CK), jnp.float32),
        ],
    )(x.reshape(B, C, NPIX), conv_w, conv_b.reshape(3, 1))


def kernel(x, conv_w, conv_b):
    offset, dest_full, lin, xw0, xw1 = _tc1(x, conv_w, conv_b)
    offset = offset.reshape(B, 2, H, W)
    dest_full = dest_full.reshape(B, C, H, W)
    idx = lin.reshape(B, NPIX)
    acc0 = jnp.zeros((B, NDST, CK), jnp.float32)
    acc1 = jnp.zeros((B, NDST, CK), jnp.float32)
    biy = jnp.arange(B)[:, None]
    acc0 = acc0.at[biy, idx].add(xw0)
    acc1 = acc1.at[biy, idx].add(xw1)
    feat = jnp.concatenate([acc0, acc1[:, :, :C - CK]], axis=2)
    att_acc = acc1[:, :, C - CK] + EPS
    out = (feat / att_acc[:, :, None]).transpose(0, 2, 1).reshape(B, C, DH, DW)
    return (out, offset, dest_full)


# speedup vs baseline: 32.1081x; 1.6942x over previous
import functools

import jax
import jax.numpy as jnp
from jax import lax
from jax.experimental import pallas as pl
from jax.experimental.pallas import tpu as pltpu

EPS = 1e-05
B, C, H, W = 2, 96, 384, 384
DH, DW = 192, 192
NPIX = H * W
NDST = DH * DW
HB = 16
PB = HB * W
NHB = H // HB
CT = 104


def _tc1_body(x_ref, w_ref, b_ref, off_ref, dest_ref, lin_ref, xw_ref):
    b = pl.program_id(0)
    h = pl.program_id(1)
    x2 = x_ref[0]
    w = w_ref[...]
    oa = lax.dot_general(w, x2, (((1,), (0,)), ((), ())),
                         preferred_element_type=jnp.float32)
    oa = oa + b_ref[...]
    pix = h * PB + lax.broadcasted_iota(jnp.int32, (1, PB), 1)
    rows = (pix // W).astype(jnp.float32)
    cols = (pix % W).astype(jnp.float32)
    gy = rows / jnp.float32(H)
    gx = cols / jnp.float32(W)
    dy = jnp.floor(jnp.clip(gy + oa[0:1], 0.0, 1.0 - EPS) * DH)
    dx = jnp.floor(jnp.clip(gx + oa[1:2], 0.0, 1.0 - EPS) * DW)
    lin = dy * DW + dx
    off_ref[0] = oa[0:2]
    chan = lax.broadcasted_iota(jnp.int32, (C, 1), 0).astype(jnp.float32) * jnp.float32(NDST)
    dest_ref[0] = lin + chan + b.astype(jnp.float32) * jnp.float32(C * NDST)
    lin_ref[0] = lin.astype(jnp.int32)
    att = jnp.exp(oa[2:3])
    xw_full = jnp.concatenate(
        [x2 * att, att, jnp.zeros((CT - C - 1, PB), jnp.float32)], axis=0)
    xw_ref[0] = xw_full.T


@jax.jit
def _tc1(x, conv_w, conv_b):
    return pl.pallas_call(
        _tc1_body,
        grid=(B, NHB),
        in_specs=[
            pl.BlockSpec((1, C, PB), lambda b, h: (b, 0, h)),
            pl.BlockSpec((3, C), lambda b, h: (0, 0)),
            pl.BlockSpec((3, 1), lambda b, h: (0, 0)),
        ],
        out_specs=[
            pl.BlockSpec((1, 2, PB), lambda b, h: (b, 0, h)),
            pl.BlockSpec((1, C, PB), lambda b, h: (b, 0, h)),
            pl.BlockSpec((1, 1, PB), lambda b, h: (b * NHB + h, 0, 0)),
            pl.BlockSpec((1, PB, CT), lambda b, h: (b, h, 0)),
        ],
        out_shape=[
            jax.ShapeDtypeStruct((B, 2, NPIX), jnp.float32),
            jax.ShapeDtypeStruct((B, C, NPIX), jnp.float32),
            jax.ShapeDtypeStruct((B * NHB, 1, PB), jnp.int32),
            jax.ShapeDtypeStruct((B, NPIX, CT), jnp.float32),
        ],
    )(x.reshape(B, C, NPIX), conv_w, conv_b.reshape(3, 1))


PBS = 1024


def _tc3_body(lin_ref, xw_ref, acc_ref):
    p = pl.program_id(1)

    @pl.when(p == 0)
    def _():
        acc_ref[...] = jnp.zeros_like(acc_ref)

    def body(i, _):
        idx = lin_ref[0, 0, i]
        acc_ref[0, pl.ds(idx, 1), :] += xw_ref[0, pl.ds(i, 1), :]
        return 0

    lax.fori_loop(0, PBS, body, 0, unroll=8)


@jax.jit
def _tc3(lin, xw):
    return pl.pallas_call(
        _tc3_body,
        grid=(B, NPIX // PBS),
        in_specs=[
            pl.BlockSpec((1, 1, PBS), lambda b, p: (b * (NPIX // PBS) + p, 0, 0),
                         memory_space=pltpu.SMEM),
            pl.BlockSpec((1, PBS, CT), lambda b, p: (b, p, 0)),
        ],
        out_specs=pl.BlockSpec((1, NDST, CT), lambda b, p: (b, 0, 0)),
        out_shape=jax.ShapeDtypeStruct((B, NDST, CT), jnp.float32),
        compiler_params=pltpu.CompilerParams(
            dimension_semantics=("arbitrary", "arbitrary")),
    )(lin, xw)


PBF = 4608


def _tc2_body(a_ref, out_ref):
    a = a_ref[0]
    att = a[:, C:C + 1] + EPS
    out_ref[0] = (a[:, :C] / att).T


@jax.jit
def _pipeline(x, conv_w, conv_b):
    offset, dest_full, lin, xw = _tc1(x, conv_w, conv_b)
    acc = _tc3(lin.reshape(B * NPIX // PBS, 1, PBS), xw)
    out = pl.pallas_call(
        _tc2_body,
        grid=(B, NDST // PBF),
        in_specs=[pl.BlockSpec((1, PBF, CT), lambda b, p: (b, p, 0))],
        out_specs=pl.BlockSpec((1, C, PBF), lambda b, p: (b, 0, p)),
        out_shape=jax.ShapeDtypeStruct((B, C, NDST), jnp.float32),
    )(acc)
    return (out.reshape(B, C, DH, DW),
            offset.reshape(B, 2, H, W),
            dest_full.reshape(B, C, H, W))


def kernel(x, conv_w, conv_b):
    return _pipeline(x, conv_w, conv_b)


# speedup vs baseline: 32.3909x; 1.0088x over previous
import functools

import jax
import jax.numpy as jnp
from jax import lax
from jax.experimental import pallas as pl
from jax.experimental.pallas import tpu as pltpu

EPS = 1e-05
B, C, H, W = 2, 96, 384, 384
DH, DW = 192, 192
NPIX = H * W
NDST = DH * DW
HB = 16
PB = HB * W
NHB = H // HB
CT = 104


def _tc1_body(x_ref, w_ref, b_ref, off_ref, dest_ref, lin_ref, xw_ref):
    b = pl.program_id(0)
    h = pl.program_id(1)
    x2 = x_ref[0]
    w = w_ref[...]
    oa = lax.dot_general(w, x2, (((1,), (0,)), ((), ())),
                         preferred_element_type=jnp.float32)
    oa = oa + b_ref[...]
    pix = h * PB + lax.broadcasted_iota(jnp.int32, (1, PB), 1)
    rows = (pix // W).astype(jnp.float32)
    cols = (pix % W).astype(jnp.float32)
    gy = rows / jnp.float32(H)
    gx = cols / jnp.float32(W)
    dy = jnp.floor(jnp.clip(gy + oa[0:1], 0.0, 1.0 - EPS) * DH)
    dx = jnp.floor(jnp.clip(gx + oa[1:2], 0.0, 1.0 - EPS) * DW)
    lin = dy * DW + dx
    off_ref[0] = oa[0:2]
    chan = lax.broadcasted_iota(jnp.int32, (C, 1), 0).astype(jnp.float32) * jnp.float32(NDST)
    dest_ref[0] = lin + chan + b.astype(jnp.float32) * jnp.float32(C * NDST)
    lin_ref[0] = lin.astype(jnp.int32)
    att = jnp.exp(oa[2:3])
    xw_full = jnp.concatenate(
        [x2 * att, att, jnp.zeros((CT - C - 1, PB), jnp.float32)], axis=0)
    xw_ref[0] = xw_full.T


@jax.jit
def _tc1(x, conv_w, conv_b):
    return pl.pallas_call(
        _tc1_body,
        grid=(B, NHB),
        in_specs=[
            pl.BlockSpec((1, C, PB), lambda b, h: (b, 0, h)),
            pl.BlockSpec((3, C), lambda b, h: (0, 0)),
            pl.BlockSpec((3, 1), lambda b, h: (0, 0)),
        ],
        out_specs=[
            pl.BlockSpec((1, 2, PB), lambda b, h: (b, 0, h)),
            pl.BlockSpec((1, C, PB), lambda b, h: (b, 0, h)),
            pl.BlockSpec((1, 1, PB), lambda b, h: (b * NHB + h, 0, 0)),
            pl.BlockSpec((1, PB, CT), lambda b, h: (b, h, 0)),
        ],
        out_shape=[
            jax.ShapeDtypeStruct((B, 2, NPIX), jnp.float32),
            jax.ShapeDtypeStruct((B, C, NPIX), jnp.float32),
            jax.ShapeDtypeStruct((B * NHB, 1, PB), jnp.int32),
            jax.ShapeDtypeStruct((B, NPIX, CT), jnp.float32),
        ],
    )(x.reshape(B, C, NPIX), conv_w, conv_b.reshape(3, 1))


PBS = 4096


def _tc3_body(lin_ref, xw_ref, acc_ref):
    p = pl.program_id(1)

    @pl.when(p == 0)
    def _():
        acc_ref[...] = jnp.zeros_like(acc_ref)

    def body(i, _):
        idx = lin_ref[0, 0, i]
        acc_ref[0, pl.ds(idx, 1), :] += xw_ref[0, pl.ds(i, 1), :]
        return 0

    lax.fori_loop(0, PBS, body, 0, unroll=8)


@jax.jit
def _tc3(lin, xw):
    return pl.pallas_call(
        _tc3_body,
        grid=(B, NPIX // PBS),
        in_specs=[
            pl.BlockSpec((1, 1, PBS), lambda b, p: (b * (NPIX // PBS) + p, 0, 0),
                         memory_space=pltpu.SMEM),
            pl.BlockSpec((1, PBS, CT), lambda b, p: (b, p, 0)),
        ],
        out_specs=pl.BlockSpec((1, NDST, CT), lambda b, p: (b, 0, 0)),
        out_shape=jax.ShapeDtypeStruct((B, NDST, CT), jnp.float32),
        compiler_params=pltpu.CompilerParams(
            dimension_semantics=("arbitrary", "arbitrary")),
    )(lin, xw)


PBF = 4608


def _tc2_body(a_ref, out_ref):
    a = a_ref[0]
    att = a[:, C:C + 1] + EPS
    out_ref[0] = (a[:, :C] / att).T


@jax.jit
def _pipeline(x, conv_w, conv_b):
    offset, dest_full, lin, xw = _tc1(x, conv_w, conv_b)
    acc = _tc3(lin.reshape(B * NPIX // PBS, 1, PBS), xw)
    out = pl.pallas_call(
        _tc2_body,
        grid=(B, NDST // PBF),
        in_specs=[pl.BlockSpec((1, PBF, CT), lambda b, p: (b, p, 0))],
        out_specs=pl.BlockSpec((1, C, PBF), lambda b, p: (b, 0, p)),
        out_shape=jax.ShapeDtypeStruct((B, C, NDST), jnp.float32),
    )(acc)
    return (out.reshape(B, C, DH, DW),
            offset.reshape(B, 2, H, W),
            dest_full.reshape(B, C, H, W))


def kernel(x, conv_w, conv_b):
    return _pipeline(x, conv_w, conv_b)
